# baseline (speedup 1.0000x reference)
import jax
import jax.numpy as jnp
from jax.experimental import pallas as pl
from jax.experimental.pallas import tpu as pltpu

OBS_DIM = 512
ACT_DIM = 64
IN_DIM = OBS_DIM + ACT_DIM
HIDDEN = 1024
CAPACITY = 16384
TOP_K = 32
BATCH = 1024

BK = 1024
NCK = CAPACITY // BK
BR = 128
NRB = BATCH // BR

_PREC = jax.lax.Precision.DEFAULT


def _scores_body(inpt_ref, w_ref, b_ref, g_ref, beta_ref, k1_ref, k2_ref,
                 s1_ref, s2_ref, phi_s):
    c = pl.program_id(0)

    @pl.when(c == 0)
    def _trunk():
        x = inpt_ref[...]
        h = jax.lax.dot_general(x, w_ref[...], (((1,), (0,)), ((), ())),
                                precision=_PREC) + b_ref[...]
        mu = jnp.mean(h, axis=1, keepdims=True)
        hc = h - mu
        var = jnp.mean(hc * hc, axis=1, keepdims=True)
        hn = hc / jnp.sqrt(var + 1e-5) * g_ref[...] + beta_ref[...]
        phi_s[...] = 2.0 * jnp.tanh(hn)

    phi = phi_s[...]
    for k_ref, s_ref in ((k1_ref, s1_ref), (k2_ref, s2_ref)):
        kc = k_ref[...]
        dot = jax.lax.dot_general(phi, kc, (((1,), (1,)), ((), ())),
                                  precision=_PREC)
        s_ref[...] = dot - jnp.sum(kc * kc, axis=1)[None, :]


def _select_body(s_ref, v_ref, q_ref):
    s = s_ref[...]
    cm = s[:, 0:128]
    cm2 = jnp.full((BR, 128), -jnp.inf, jnp.float32)
    for j in range(1, CAPACITY // 128):
        tj = s[:, j * 128:(j + 1) * 128]
        lo = jnp.minimum(cm, tj)
        cm = jnp.maximum(cm, tj)
        cm2 = jnp.maximum(cm2, lo)
    last = jnp.full((BR, 1), jnp.inf, jnp.float32)
    mx = None
    for i in range(TOP_K):
        c1 = jnp.where(cm < last, cm, -jnp.inf)
        c2 = jnp.where(cm2 < last, cm2, -jnp.inf)
        last = jnp.max(jnp.maximum(c1, c2), axis=1, keepdims=True)
        if i == 0:
            mx = last
    need = (jnp.sum(jnp.where(s > last, 1.0, 0.0),
                    axis=1, keepdims=True) - 31.0)

    def _cond(carry):
        _, nd = carry
        return jnp.max(nd) > 0.0

    def _refine(carry):
        lst, nd = carry
        nm = jnp.min(jnp.where(s > lst, s, jnp.inf), axis=1, keepdims=True)
        pred = nd > 0.0
        return (jnp.where(pred, nm, lst), nd - jnp.where(pred, 1.0, 0.0))

    t, _ = jax.lax.while_loop(_cond, _refine, (last, need))
    w = jnp.where(s >= t, jnp.exp(s - mx), 0.0)
    den = jnp.sum(w, axis=1, keepdims=True)
    num = jnp.sum(w * v_ref[...], axis=1, keepdims=True)
    q_ref[...] = jnp.broadcast_to(num / den, (BR, 128))


def _scores(inpt, W_trunk, b_trunk, ln_g, ln_b, keys1, keys2):
    return pl.pallas_call(
        _scores_body,
        grid=(NCK,),
        in_specs=[
            pl.BlockSpec((BATCH, IN_DIM), lambda c: (0, 0)),
            pl.BlockSpec((IN_DIM, HIDDEN), lambda c: (0, 0)),
            pl.BlockSpec((1, HIDDEN), lambda c: (0, 0)),
            pl.BlockSpec((1, HIDDEN), lambda c: (0, 0)),
            pl.BlockSpec((1, HIDDEN), lambda c: (0, 0)),
            pl.BlockSpec((BK, HIDDEN), lambda c: (c, 0)),
            pl.BlockSpec((BK, HIDDEN), lambda c: (c, 0)),
        ],
        out_specs=[
            pl.BlockSpec((BATCH, BK), lambda c: (0, c)),
            pl.BlockSpec((BATCH, BK), lambda c: (0, c)),
        ],
        out_shape=[
            jax.ShapeDtypeStruct((BATCH, CAPACITY), jnp.float32),
            jax.ShapeDtypeStruct((BATCH, CAPACITY), jnp.float32),
        ],
        scratch_shapes=[pltpu.VMEM((BATCH, HIDDEN), jnp.float32)],
        compiler_params=pltpu.CompilerParams(
            dimension_semantics=("arbitrary",),
        ),
    )(inpt, W_trunk, b_trunk.reshape(1, HIDDEN), ln_g.reshape(1, HIDDEN),
      ln_b.reshape(1, HIDDEN), keys1, keys2)


def _select(s, vt):
    return pl.pallas_call(
        _select_body,
        grid=(NRB,),
        in_specs=[
            pl.BlockSpec((BR, CAPACITY), lambda r: (r, 0)),
            pl.BlockSpec((1, CAPACITY), lambda r: (0, 0)),
        ],
        out_specs=pl.BlockSpec((BR, 128), lambda r: (r, 0)),
        out_shape=jax.ShapeDtypeStruct((BATCH, 128), jnp.float32),
        compiler_params=pltpu.CompilerParams(
            dimension_semantics=("parallel",),
        ),
    )(s, vt)


@jax.jit
def kernel(obs, action, W_trunk, b_trunk, ln_g, ln_b,
           keys1, values1, keys2, values2):
    inpt = jnp.concatenate([obs, action], axis=-1)
    s1, s2 = _scores(inpt, W_trunk, b_trunk, ln_g, ln_b, keys1, keys2)
    q1 = _select(s1, values1.reshape(1, CAPACITY))
    q2 = _select(s2, values2.reshape(1, CAPACITY))
    return (q1[:, :1], q2[:, :1])


# speedup vs baseline: 1.0113x; 1.0113x over previous
import jax
import jax.numpy as jnp
from jax.experimental import pallas as pl
from jax.experimental.pallas import tpu as pltpu

OBS_DIM = 512
ACT_DIM = 64
IN_DIM = OBS_DIM + ACT_DIM
HIDDEN = 1024
CAPACITY = 16384
TOP_K = 32
BATCH = 1024

BK = 1024
NCK = CAPACITY // BK
BR = 128
NRB = BATCH // BR

_PREC = jax.lax.Precision.DEFAULT


def _scores_body(inpt_ref, w_ref, b_ref, g_ref, beta_ref, k1_ref, k2_ref,
                 s1_ref, s2_ref, phi_s):
    c = pl.program_id(0)

    @pl.when(c == 0)
    def _trunk():
        x = inpt_ref[...]
        h = jax.lax.dot_general(x, w_ref[...], (((1,), (0,)), ((), ())),
                                precision=_PREC) + b_ref[...]
        mu = jnp.mean(h, axis=1, keepdims=True)
        hc = h - mu
        var = jnp.mean(hc * hc, axis=1, keepdims=True)
        hn = hc / jnp.sqrt(var + 1e-5) * g_ref[...] + beta_ref[...]
        phi_s[...] = 2.0 * jnp.tanh(hn)

    phi = phi_s[...]
    for k_ref, s_ref in ((k1_ref, s1_ref), (k2_ref, s2_ref)):
        kc = k_ref[...]
        dot = jax.lax.dot_general(phi, kc, (((1,), (1,)), ((), ())),
                                  precision=_PREC)
        s_ref[...] = dot - jnp.sum(kc * kc, axis=1)[None, :]


def _select_body(s_ref, v_ref, q_ref):
    s = s_ref[...]
    cm = s[:, 0:128]
    cm2 = jnp.full((BR, 128), -jnp.inf, jnp.float32)
    for j in range(1, CAPACITY // 128):
        tj = s[:, j * 128:(j + 1) * 128]
        lo = jnp.minimum(cm, tj)
        cm = jnp.maximum(cm, tj)
        cm2 = jnp.maximum(cm2, lo)
    last = jnp.full((BR, 1), jnp.inf, jnp.float32)
    mx = None
    for i in range(TOP_K):
        c1 = jnp.where(cm < last, cm, -jnp.inf)
        c2 = jnp.where(cm2 < last, cm2, -jnp.inf)
        last = jnp.max(jnp.maximum(c1, c2), axis=1, keepdims=True)
        if i == 0:
            mx = last
    gt = s > last
    need = jnp.sum(jnp.where(gt, 1.0, 0.0), axis=1, keepdims=True) - 31.0
    nm0 = jnp.min(jnp.where(gt, s, jnp.inf), axis=1, keepdims=True)
    pred0 = need > 0.0
    last = jnp.where(pred0, nm0, last)
    need = need - jnp.where(pred0, 1.0, 0.0)

    def _cond(carry):
        _, nd = carry
        return jnp.max(nd) > 0.0

    def _refine(carry):
        lst, nd = carry
        nm = jnp.min(jnp.where(s > lst, s, jnp.inf), axis=1, keepdims=True)
        pred = nd > 0.0
        return (jnp.where(pred, nm, lst), nd - jnp.where(pred, 1.0, 0.0))

    t, _ = jax.lax.while_loop(_cond, _refine, (last, need))
    w = jnp.where(s >= t, jnp.exp(s - mx), 0.0)
    den = jnp.sum(w, axis=1, keepdims=True)
    num = jnp.sum(w * v_ref[...], axis=1, keepdims=True)
    q_ref[...] = jnp.broadcast_to(num / den, (BR, 128))


def _scores(inpt, W_trunk, b_trunk, ln_g, ln_b, keys1, keys2):
    return pl.pallas_call(
        _scores_body,
        grid=(NCK,),
        in_specs=[
            pl.BlockSpec((BATCH, IN_DIM), lambda c: (0, 0)),
            pl.BlockSpec((IN_DIM, HIDDEN), lambda c: (0, 0)),
            pl.BlockSpec((1, HIDDEN), lambda c: (0, 0)),
            pl.BlockSpec((1, HIDDEN), lambda c: (0, 0)),
            pl.BlockSpec((1, HIDDEN), lambda c: (0, 0)),
            pl.BlockSpec((BK, HIDDEN), lambda c: (c, 0)),
            pl.BlockSpec((BK, HIDDEN), lambda c: (c, 0)),
        ],
        out_specs=[
            pl.BlockSpec((BATCH, BK), lambda c: (0, c)),
            pl.BlockSpec((BATCH, BK), lambda c: (0, c)),
        ],
        out_shape=[
            jax.ShapeDtypeStruct((BATCH, CAPACITY), jnp.float32),
            jax.ShapeDtypeStruct((BATCH, CAPACITY), jnp.float32),
        ],
        scratch_shapes=[pltpu.VMEM((BATCH, HIDDEN), jnp.float32)],
        compiler_params=pltpu.CompilerParams(
            dimension_semantics=("arbitrary",),
        ),
    )(inpt, W_trunk, b_trunk.reshape(1, HIDDEN), ln_g.reshape(1, HIDDEN),
      ln_b.reshape(1, HIDDEN), keys1, keys2)


def _select(s, vt):
    return pl.pallas_call(
        _select_body,
        grid=(NRB,),
        in_specs=[
            pl.BlockSpec((BR, CAPACITY), lambda r: (r, 0)),
            pl.BlockSpec((1, CAPACITY), lambda r: (0, 0)),
        ],
        out_specs=pl.BlockSpec((BR, 128), lambda r: (r, 0)),
        out_shape=jax.ShapeDtypeStruct((BATCH, 128), jnp.float32),
        compiler_params=pltpu.CompilerParams(
            dimension_semantics=("parallel",),
        ),
    )(s, vt)


@jax.jit
def kernel(obs, action, W_trunk, b_trunk, ln_g, ln_b,
           keys1, values1, keys2, values2):
    inpt = jnp.concatenate([obs, action], axis=-1)
    s1, s2 = _scores(inpt, W_trunk, b_trunk, ln_g, ln_b, keys1, keys2)
    q1 = _select(s1, values1.reshape(1, CAPACITY))
    q2 = _select(s2, values2.reshape(1, CAPACITY))
    return (q1[:, :1], q2[:, :1])
